# baseline (speedup 1.0000x reference)
import functools

import jax
import jax.numpy as jnp
from jax import lax
from jax.experimental import pallas as pl
from jax.experimental.pallas import tpu as pltpu
from jax.experimental.pallas import tpu_sc as plsc

VOCAB = 1000000
D = 32
S = 50
B = 4096
N = B * S
V4 = VOCAB // 4
NC, NS = 2, 16
NW = NC * NS
PER_W = N // NW
CH = 800
NCHUNK = PER_W // CH

DT_R = 2000

_mesh = plsc.VectorSubcoreMesh(core_axis_name="c", subcore_axis_name="s")


def _dt_body(t0_ref, t1_ref, t2_ref, t3_ref, o_ref):
    o_ref[...] = jnp.concatenate(
        [t0_ref[...], t1_ref[...], t2_ref[...], t3_ref[...]], axis=1
    )


def _make_dt_spec(k):
    nb = V4 // DT_R
    return pl.BlockSpec((DT_R, D), lambda i, _k=k, _nb=nb: (_k * _nb + i, 0))


def _tc_detile(table):
    return pl.pallas_call(
        _dt_body,
        grid=(V4 // DT_R,),
        in_specs=[_make_dt_spec(k) for k in range(4)],
        out_specs=pl.BlockSpec((DT_R, 128), lambda i: (i, 0)),
        out_shape=jax.ShapeDtypeStruct((V4, 128), jnp.float32),
    )(table, table, table, table)


@functools.partial(
    pl.kernel,
    mesh=_mesh,
    out_type=jax.ShapeDtypeStruct((N, 128), jnp.float32),
    scratch_types=[
        pltpu.VMEM((PER_W,), jnp.int32),
        pltpu.VMEM((CH, 128), jnp.float32),
        pltpu.SemaphoreType.DMA,
    ],
)
def _sc_gather(t128_hbm, idx_hbm, out_hbm, idx_v, rows_v, sem):
    wid = lax.axis_index("s") * NC + lax.axis_index("c")
    base = wid * PER_W
    pltpu.sync_copy(idx_hbm.at[pl.ds(base, PER_W)], idx_v)
    for i in range(NCHUNK):
        off = i * CH
        pltpu.async_copy(
            t128_hbm.at[idx_v.at[pl.ds(off, CH)]], rows_v, sem
        ).wait()
        pltpu.sync_copy(rows_v, out_hbm.at[pl.ds(base + off, CH)])


_BB = 512


def _mm_body(g_ref, k_ref, w_ref, b_ref, o_ref):
    s = pl.program_id(1)

    @pl.when(s == 0)
    def _():
        o_ref[...] = jnp.broadcast_to(b_ref[...], (_BB, D))

    sel = (lax.broadcasted_iota(jnp.int32, (1, S), 1) == s).astype(jnp.int32)
    ks = jnp.sum(k_ref[...] * sel, axis=1, keepdims=True)
    grp = lax.broadcasted_iota(jnp.int32, (_BB, 128), 1) >> 5
    e = jnp.where(grp == ks, g_ref[...], 0.0)
    o_ref[...] += lax.dot_general(
        e, w_ref[0],
        (((1,), (0,)), ((), ())),
        preferred_element_type=jnp.float32,
    )


def _tc_matmul(g, k, w4, b):
    return pl.pallas_call(
        _mm_body,
        grid=(B // _BB, S),
        in_specs=[
            pl.BlockSpec((_BB, 128), lambda i, s: (s * (B // _BB) + i, 0)),
            pl.BlockSpec((_BB, S), lambda i, s: (i, 0)),
            pl.BlockSpec((1, 128, D), lambda i, s: (s, 0, 0)),
            pl.BlockSpec((1, D), lambda i, s: (0, 0)),
        ],
        out_specs=pl.BlockSpec((_BB, D), lambda i, s: (i, 0)),
        out_shape=jax.ShapeDtypeStruct((B, D), jnp.float32),
    )(g, k, w4, b)


def kernel(x, table, W, b):
    xi = x.astype(jnp.int32)
    idx4 = (xi.T % V4).reshape(N)
    k = xi // V4
    t128 = _tc_detile(table)
    g = _sc_gather(t128, idx4)
    w4 = jnp.tile(W.T.reshape(S, D, D), (1, 4, 1))
    return _tc_matmul(g, k, w4, b.reshape(1, D))


# speedup vs baseline: 1.2842x; 1.2842x over previous
import functools

import jax
import jax.numpy as jnp
from jax import lax
from jax.experimental import pallas as pl
from jax.experimental.pallas import tpu as pltpu
from jax.experimental.pallas import tpu_sc as plsc

VOCAB = 1000000
D = 32
S = 50
B = 4096
N = B * S
V4 = VOCAB // 4
NC, NS = 2, 16
NW = NC * NS
PER_W = N // NW
CH = 800
NCHUNK = PER_W // CH

DT_R = 10000

_mesh = plsc.VectorSubcoreMesh(core_axis_name="c", subcore_axis_name="s")


def _dt_body(t_ref, o_ref):
    q = pl.program_id(1)
    lane = lax.broadcasted_iota(jnp.int32, (D, 128), 1)
    sub = lax.broadcasted_iota(jnp.int32, (D, 128), 0)
    e = (lane - 32 * q == sub).astype(jnp.float32)
    prod = lax.dot_general(
        t_ref[...], e,
        (((1,), (0,)), ((), ())),
        preferred_element_type=jnp.float32,
    )

    @pl.when(q == 0)
    def _():
        o_ref[...] = prod

    @pl.when(q != 0)
    def _():
        o_ref[...] += prod


def _tc_detile(table):
    nb = V4 // DT_R
    return pl.pallas_call(
        _dt_body,
        grid=(nb, 4),
        in_specs=[pl.BlockSpec((DT_R, D), lambda i, q: (q * nb + i, 0))],
        out_specs=pl.BlockSpec((DT_R, 128), lambda i, q: (i, 0)),
        out_shape=jax.ShapeDtypeStruct((V4, 128), jnp.float32),
    )(table)


@functools.partial(
    pl.kernel,
    mesh=_mesh,
    out_type=jax.ShapeDtypeStruct((N, 128), jnp.float32),
    scratch_types=[
        pltpu.VMEM((PER_W,), jnp.int32),
        pltpu.VMEM((CH, 128), jnp.float32),
        pltpu.SemaphoreType.DMA,
    ],
)
def _sc_gather(t128_hbm, idx_hbm, out_hbm, idx_v, rows_v, sem):
    wid = lax.axis_index("s") * NC + lax.axis_index("c")
    base = wid * PER_W
    pltpu.sync_copy(idx_hbm.at[pl.ds(base, PER_W)], idx_v)
    for i in range(NCHUNK):
        off = i * CH
        pltpu.async_copy(
            t128_hbm.at[idx_v.at[pl.ds(off, CH)]], rows_v, sem
        ).wait()
        pltpu.sync_copy(rows_v, out_hbm.at[pl.ds(base + off, CH)])


def _mm_body(g_ref, kf_ref, w_ref, b_ref, o_ref):
    s = pl.program_id(0)

    @pl.when(s == 0)
    def _():
        o_ref[...] = jnp.broadcast_to(b_ref[...], (B, D))

    oh = (lax.broadcasted_iota(jnp.int32, (S, 1), 0) == s).astype(jnp.float32)
    ks = lax.dot_general(
        kf_ref[...], oh,
        (((1,), (0,)), ((), ())),
        preferred_element_type=jnp.float32,
    ).astype(jnp.int32)
    grp = lax.broadcasted_iota(jnp.int32, (B, 128), 1) >> 5
    e = jnp.where(grp == ks, g_ref[...], 0.0)
    o_ref[...] += lax.dot_general(
        e, w_ref[0],
        (((1,), (0,)), ((), ())),
        preferred_element_type=jnp.float32,
    )


def _tc_matmul(g, kf, w4, b):
    return pl.pallas_call(
        _mm_body,
        grid=(S,),
        in_specs=[
            pl.BlockSpec((B, 128), lambda s: (s, 0)),
            pl.BlockSpec((B, S), lambda s: (0, 0)),
            pl.BlockSpec((1, 128, D), lambda s: (s, 0, 0)),
            pl.BlockSpec((1, D), lambda s: (0, 0)),
        ],
        out_specs=pl.BlockSpec((B, D), lambda s: (0, 0)),
        out_shape=jax.ShapeDtypeStruct((B, D), jnp.float32),
    )(g, kf, w4, b)


def kernel(x, table, W, b):
    xi = x.astype(jnp.int32)
    idx4 = (xi.T % V4).reshape(N)
    kf = (xi // V4).astype(jnp.float32)
    t128 = _tc_detile(table)
    g = _sc_gather(t128, idx4)
    w4 = jnp.tile(W.T.reshape(S, D, D), (1, 4, 1))
    return _tc_matmul(g, kf, w4, b.reshape(1, D))


# speedup vs baseline: 1.3539x; 1.0542x over previous
import functools

import jax
import jax.numpy as jnp
from jax import lax
from jax.experimental import pallas as pl
from jax.experimental.pallas import tpu as pltpu
from jax.experimental.pallas import tpu_sc as plsc

VOCAB = 1000000
D = 32
S = 50
B = 4096
N = B * S
V4 = VOCAB // 4
NC, NS = 2, 16
NW = NC * NS
PER_W = N // NW
CH = 800
NCHUNK = PER_W // CH

DT_R = 10000

_mesh = plsc.VectorSubcoreMesh(core_axis_name="c", subcore_axis_name="s")


def _dt_body(t_ref, o_ref):
    q = pl.program_id(1)
    lane = lax.broadcasted_iota(jnp.int32, (D, 128), 1)
    sub = lax.broadcasted_iota(jnp.int32, (D, 128), 0)
    e = (lane - 32 * q == sub).astype(jnp.float32)
    prod = lax.dot_general(
        t_ref[...], e,
        (((1,), (0,)), ((), ())),
        preferred_element_type=jnp.float32,
    )

    @pl.when(q == 0)
    def _():
        o_ref[...] = prod

    @pl.when(q != 0)
    def _():
        o_ref[...] += prod


def _tc_detile(table):
    nb = V4 // DT_R
    return pl.pallas_call(
        _dt_body,
        grid=(nb, 4),
        in_specs=[pl.BlockSpec((DT_R, D), lambda i, q: (q * nb + i, 0))],
        out_specs=pl.BlockSpec((DT_R, 128), lambda i, q: (i, 0)),
        out_shape=jax.ShapeDtypeStruct((V4, 128), jnp.float32),
    )(table)


@functools.partial(
    pl.kernel,
    mesh=_mesh,
    out_type=jax.ShapeDtypeStruct((N, 128), jnp.float32),
    scratch_types=[
        pltpu.VMEM((PER_W,), jnp.int32),
        pltpu.VMEM((CH, 128), jnp.float32),
        pltpu.SemaphoreType.DMA,
    ],
)
def _sc_gather(t128_hbm, idx_hbm, out_hbm, idx_v, rows_v, sem):
    wid = lax.axis_index("s") * NC + lax.axis_index("c")
    base = wid * PER_W
    pltpu.sync_copy(idx_hbm.at[pl.ds(base, PER_W)], idx_v)
    for i in range(NCHUNK):
        off = i * CH
        pltpu.async_copy(
            t128_hbm.at[idx_v.at[pl.ds(off, CH)]], rows_v, sem
        ).wait()
        pltpu.sync_copy(rows_v, out_hbm.at[pl.ds(base + off, CH)])


def _mm_body(g_ref, kf_ref, w_ref, b_ref, o_ref):
    s = pl.program_id(0)

    @pl.when(s == 0)
    def _():
        o_ref[...] = jnp.broadcast_to(b_ref[...], (B, D))

    oh = (lax.broadcasted_iota(jnp.int32, (S, 1), 0) == s).astype(jnp.float32)
    ks = lax.dot_general(
        kf_ref[...], oh,
        (((1,), (0,)), ((), ())),
        preferred_element_type=jnp.float32,
    ).astype(jnp.int32)
    grp = lax.broadcasted_iota(jnp.int32, (B, 128), 1) >> 5
    e = jnp.where(grp == ks, g_ref[...], 0.0)
    o_ref[...] += lax.dot_general(
        e, w_ref[0],
        (((1,), (0,)), ((), ())),
        preferred_element_type=jnp.float32,
    )


def _tc_matmul(g, kf, w4, b):
    return pl.pallas_call(
        _mm_body,
        grid=(S,),
        in_specs=[
            pl.BlockSpec((B, 128), lambda s: (s, 0)),
            pl.BlockSpec((B, S), lambda s: (0, 0)),
            pl.BlockSpec((1, 128, D), lambda s: (s, 0, 0)),
            pl.BlockSpec((1, D), lambda s: (0, 0)),
        ],
        out_specs=pl.BlockSpec((B, D), lambda s: (0, 0)),
        out_shape=jax.ShapeDtypeStruct((B, D), jnp.float32),
    )(g, kf, w4, b)


def kernel(x, table, W, b):
    xi = x.astype(jnp.int32)
    idx4 = (xi.T // 4).reshape(N)
    kf = (xi % 4).astype(jnp.float32)
    t128 = jnp.reshape(table, (V4, 128))
    g = _sc_gather(t128, idx4)
    w4 = jnp.tile(W.T.reshape(S, D, D), (1, 4, 1))
    return _tc_matmul(g, kf, w4, b.reshape(1, D))


# speedup vs baseline: 1.5493x; 1.1444x over previous
import functools

import jax
import jax.numpy as jnp
from jax import lax
from jax.experimental import pallas as pl
from jax.experimental.pallas import tpu as pltpu
from jax.experimental.pallas import tpu_sc as plsc

VOCAB = 1000000
D = 32
S = 50
B = 4096
N = B * S
NC, NS = 2, 16
NW = NC * NS
PER_W = N // NW
CH = 1600
NCHUNK = PER_W // CH
BP = B // 4

_mesh = plsc.VectorSubcoreMesh(core_axis_name="c", subcore_axis_name="s")


@functools.partial(
    pl.kernel,
    mesh=_mesh,
    out_type=jax.ShapeDtypeStruct((N, D), jnp.float32),
    scratch_types=[
        pltpu.VMEM((PER_W,), jnp.int32),
        pltpu.VMEM((CH, D), jnp.float32),
        pltpu.SemaphoreType.DMA,
    ],
    compiler_params=pltpu.CompilerParams(use_tc_tiling_on_sc=False),
)
def _sc_gather(table_hbm, idx_hbm, out_hbm, idx_v, rows_v, sem):
    wid = lax.axis_index("s") * NC + lax.axis_index("c")
    base = wid * PER_W
    pltpu.sync_copy(idx_hbm.at[pl.ds(base, PER_W)], idx_v)
    for i in range(NCHUNK):
        off = i * CH
        pltpu.async_copy(
            table_hbm.at[idx_v.at[pl.ds(off, CH)]], rows_v, sem
        ).wait()
        pltpu.sync_copy(rows_v, out_hbm.at[pl.ds(base + off, CH)])


def _mm_body(g_ref, w_ref, b_ref, o_ref):
    s = pl.program_id(0)

    @pl.when(s == 0)
    def _():
        o_ref[...] = jnp.broadcast_to(b_ref[...], (BP, 128))

    o_ref[...] += lax.dot_general(
        g_ref[...], w_ref[0],
        (((1,), (0,)), ((), ())),
        preferred_element_type=jnp.float32,
    )


def _tc_matmul(gp, wd, b128):
    return pl.pallas_call(
        _mm_body,
        grid=(S,),
        in_specs=[
            pl.BlockSpec((BP, 128), lambda s: (s, 0)),
            pl.BlockSpec((1, 128, 128), lambda s: (s, 0, 0)),
            pl.BlockSpec((1, 128), lambda s: (0, 0)),
        ],
        out_specs=pl.BlockSpec((BP, 128), lambda s: (0, 0)),
        out_shape=jax.ShapeDtypeStruct((BP, 128), jnp.float32),
    )(gp, wd, b128)


def kernel(x, table, W, b):
    xi = x.astype(jnp.int32)
    idx = xi.T.reshape(N)
    g = _sc_gather(table, idx)
    gp = g.reshape(BP * S, 128)
    wt = W.T.reshape(S, D, D)
    wd = jnp.einsum(
        "mn,sjo->smjno", jnp.eye(4, dtype=jnp.float32), wt
    ).reshape(S, 128, 128)
    b128 = jnp.tile(b, 4).reshape(1, 128)
    packed = _tc_matmul(gp, wd, b128)
    return packed.reshape(B, D)
